# baseline (speedup 1.0000x reference)
import functools

import jax
import jax.numpy as jnp
from jax import lax
from jax.experimental import pallas as pl
from jax.experimental.pallas import tpu as pltpu
from jax.experimental.pallas import tpu_sc as plsc

_NUM_CORES = 2
_NUM_SUBCORES = 16
_NUM_WORKERS = _NUM_CORES * _NUM_SUBCORES

_TBLK = 7


def _sc_gather(label, cls_ctx):
    b = label.shape[0]
    n_ctx, d = cls_ctx.shape[1], cls_ctx.shape[2]
    bpw = b // _NUM_WORKERS

    mesh = plsc.VectorSubcoreMesh(core_axis_name="c", subcore_axis_name="s")

    @functools.partial(
        pl.kernel,
        mesh=mesh,
        out_type=jax.ShapeDtypeStruct((b, n_ctx, d), jnp.float32),
        scratch_types=[
            pltpu.VMEM((bpw,), jnp.int32),
            pltpu.VMEM((bpw, n_ctx, d), jnp.float32),
            pltpu.SemaphoreType.DMA,
        ],
    )
    def body(label_hbm, table_hbm, out_hbm, idx_v, rows_v, sem):
        wid = lax.axis_index("s") * _NUM_CORES + lax.axis_index("c")
        base = wid * bpw
        pltpu.sync_copy(label_hbm.at[pl.ds(base, bpw)], idx_v)
        pltpu.async_copy(table_hbm.at[idx_v], rows_v, sem).wait()
        pltpu.sync_copy(rows_v, out_hbm.at[pl.ds(base, bpw)])

    return body(label, cls_ctx)


def _tc_fill_suffix(suffix_t, b, tok):
    suf, _, d = suffix_t.shape
    lead = tok - suf
    grid = (tok // _TBLK - 1,)

    def body(suf_ref, out_ref):
        i = pl.program_id(0)
        for k in range(_TBLK):
            row = suf_ref[pl.ds((i + 1) * _TBLK + k - lead, 1)]
            out_ref[k] = jnp.broadcast_to(row[0], (b, d))

    return pl.pallas_call(
        body,
        grid=grid,
        in_specs=[pl.BlockSpec((suf, 1, d), lambda i: (0, 0, 0))],
        out_specs=pl.BlockSpec((_TBLK, b, d), lambda i: (i + 1, 0, 0)),
        out_shape=jax.ShapeDtypeStruct((tok, b, d), jnp.float32),
    )(suffix_t)


def _tc_fill_head(partial, cls, token_prefix, suffix_t):
    tok, b, d = partial.shape
    n_ctx = cls.shape[1]
    pre = token_prefix.shape[1]
    lead = pre + n_ctx

    def body(cls_ref, pre_ref, suf_ref, partial_ref, out_ref):
        del partial_ref
        out_ref[0] = jnp.broadcast_to(pre_ref[0], (b, d))
        for t in range(1, lead):
            out_ref[t] = cls_ref[:, t - 1, :]
        for k in range(lead, _TBLK):
            out_ref[k] = jnp.broadcast_to(suf_ref[k - lead, 0], (b, d))

    return pl.pallas_call(
        body,
        grid=(1,),
        in_specs=[
            pl.BlockSpec((b, n_ctx, d), lambda i: (0, 0, 0)),
            pl.BlockSpec((1, pre, d), lambda i: (0, 0, 0)),
            pl.BlockSpec((_TBLK - lead, 1, d), lambda i: (0, 0, 0)),
            pl.BlockSpec(memory_space=pl.ANY),
        ],
        out_specs=pl.BlockSpec((_TBLK, b, d), lambda i: (0, 0, 0)),
        out_shape=jax.ShapeDtypeStruct((tok, b, d), jnp.float32),
        input_output_aliases={3: 0},
    )(cls, token_prefix, suffix_t, partial)


def kernel(label, cls_ctx, token_prefix, token_suffix):
    b = label.shape[0]
    tok = token_prefix.shape[1] + cls_ctx.shape[1] + token_suffix.shape[1]
    cls = _sc_gather(label, cls_ctx)
    suffix_t = jnp.transpose(token_suffix, (1, 0, 2))
    partial = _tc_fill_suffix(suffix_t, b, tok)
    out_t = _tc_fill_head(partial, cls, token_prefix, suffix_t)
    return jnp.transpose(out_t, (1, 0, 2))


# speedup vs baseline: 1.0347x; 1.0347x over previous
import functools

import jax
import jax.numpy as jnp
from jax import lax
from jax.experimental import pallas as pl
from jax.experimental.pallas import tpu as pltpu
from jax.experimental.pallas import tpu_sc as plsc

_NUM_CORES = 2
_NUM_SUBCORES = 16
_NUM_WORKERS = _NUM_CORES * _NUM_SUBCORES

_TBLK = 7


def _sc_gather(label, cls_ctx):
    b = label.shape[0]
    n_ctx, d = cls_ctx.shape[1], cls_ctx.shape[2]
    bpw = b // _NUM_WORKERS

    mesh = plsc.VectorSubcoreMesh(core_axis_name="c", subcore_axis_name="s")

    @functools.partial(
        pl.kernel,
        mesh=mesh,
        out_type=jax.ShapeDtypeStruct((b, n_ctx, d), jnp.float32),
        scratch_types=[
            pltpu.VMEM((bpw,), jnp.int32),
            pltpu.VMEM((bpw, n_ctx, d), jnp.float32),
            pltpu.SemaphoreType.DMA,
        ],
    )
    def body(label_hbm, table_hbm, out_hbm, idx_v, rows_v, sem):
        wid = lax.axis_index("s") * _NUM_CORES + lax.axis_index("c")
        base = wid * bpw
        pltpu.sync_copy(label_hbm.at[pl.ds(base, bpw)], idx_v)
        pltpu.async_copy(table_hbm.at[idx_v], rows_v, sem).wait()
        pltpu.sync_copy(rows_v, out_hbm.at[pl.ds(base, bpw)])

    return body(label, cls_ctx)


def _tc_fill_suffix(suffix_t, b, tok):
    suf, _, d = suffix_t.shape
    lead = tok - suf
    grid = (tok // _TBLK - 1,)

    def body(suf_ref, out_ref):
        i = pl.program_id(0)
        for k in range(_TBLK):
            row = suf_ref[pl.ds((i + 1) * _TBLK + k - lead, 1)]
            out_ref[k] = jnp.broadcast_to(row[0], (b, d))

    return pl.pallas_call(
        body,
        grid=grid,
        in_specs=[pl.BlockSpec((suf, 1, d), lambda i: (0, 0, 0))],
        out_specs=pl.BlockSpec((_TBLK, b, d), lambda i: (i + 1, 0, 0)),
        out_shape=jax.ShapeDtypeStruct((tok, b, d), jnp.float32),
    )(suffix_t)


def _tc_fill_head(partial, cls, token_prefix, suffix_t):
    tok, b, d = partial.shape
    n_ctx = cls.shape[1]
    pre = token_prefix.shape[1]
    lead = pre + n_ctx

    nb = 4
    bb = b // nb

    def body(cls_ref, pre_ref, suf_ref, partial_ref, out_ref):
        del partial_ref
        out_ref[0] = jnp.broadcast_to(pre_ref[0], (bb, d))
        for t in range(1, lead):
            out_ref[t] = cls_ref[:, t - 1, :]
        for k in range(lead, _TBLK):
            out_ref[k] = jnp.broadcast_to(suf_ref[k - lead, 0], (bb, d))

    return pl.pallas_call(
        body,
        grid=(nb,),
        in_specs=[
            pl.BlockSpec((bb, n_ctx, d), lambda i: (i, 0, 0)),
            pl.BlockSpec((1, pre, d), lambda i: (0, 0, 0)),
            pl.BlockSpec((_TBLK - lead, 1, d), lambda i: (0, 0, 0)),
            pl.BlockSpec(memory_space=pl.ANY),
        ],
        out_specs=pl.BlockSpec((_TBLK, bb, d), lambda i: (0, i, 0)),
        out_shape=jax.ShapeDtypeStruct((tok, b, d), jnp.float32),
        input_output_aliases={3: 0},
    )(cls, token_prefix, suffix_t, partial)


def kernel(label, cls_ctx, token_prefix, token_suffix):
    b = label.shape[0]
    tok = token_prefix.shape[1] + cls_ctx.shape[1] + token_suffix.shape[1]
    cls = _sc_gather(label, cls_ctx)
    suffix_t = jnp.transpose(token_suffix, (1, 0, 2))
    partial = _tc_fill_suffix(suffix_t, b, tok)
    out_t = _tc_fill_head(partial, cls, token_prefix, suffix_t)
    return jnp.transpose(out_t, (1, 0, 2))
